# Initial kernel scaffold; baseline (speedup 1.0000x reference)
import jax
import jax.numpy as jnp
from jax import lax
from jax.experimental import pallas as pl
from jax.experimental.pallas import tpu as pltpu
from jax.experimental.pallas import tpu_sc as plsc

N = 10000
E = 320000
F_IN = 128
HC = 64
H1 = 8
NCLS = 40

NP = 10240
NB = 16
BN = NP // NB

CH1 = 158
CH2 = 80
CK = 128
EP1 = 16 * CH1 * CK
EP2 = 32 * CH2 * CK

_f32 = jnp.float32


def _tc1_body(x_ref, w_ref, as_ref, ad_ref, h_ref, als_ref, ald_ref):
    hb = jnp.dot(x_ref[...], w_ref[...], preferred_element_type=_f32)
    h_ref[0] = hb
    als_ref[0, :] = jnp.dot(hb, as_ref[0, :], preferred_element_type=_f32)
    ald_ref[0, :] = jnp.dot(hb, ad_ref[0, :], preferred_element_type=_f32)


@jax.jit
def _tc1(xp, W1, a_src1, a_dst1):
    return pl.pallas_call(
        _tc1_body,
        grid=(NB, H1),
        in_specs=[
            pl.BlockSpec((BN, F_IN), lambda i, k: (i, 0)),
            pl.BlockSpec((F_IN, HC), lambda i, k: (0, k)),
            pl.BlockSpec((1, HC), lambda i, k: (k, 0)),
            pl.BlockSpec((1, HC), lambda i, k: (k, 0)),
        ],
        out_specs=[
            pl.BlockSpec((1, BN, HC), lambda i, k: (k, i, 0)),
            pl.BlockSpec((1, BN), lambda i, k: (k, i)),
            pl.BlockSpec((1, BN), lambda i, k: (k, i)),
        ],
        out_shape=[
            jax.ShapeDtypeStruct((H1, NP, HC), _f32),
            jax.ShapeDtypeStruct((H1, NP), _f32),
            jax.ShapeDtypeStruct((H1, NP), _f32),
        ],
    )(xp, W1, a_src1, a_dst1)


def _edge_phase(hk_s, outk_s, denk_s, src_v, dst_v, als_v, ald_v,
                rows0_v, rows1_v, p_v, row0, n_chunks, sem0, sem1):
    pltpu.async_copy(hk_s.at[src_v.at[0]], rows0_v, sem0)
    pltpu.async_copy(hk_s.at[src_v.at[1]], rows1_v, sem1)

    def pair_body(g, carry):
        for b, buf, sem in ((0, rows0_v, sem0), (1, rows1_v, sem1)):
            ci = 2 * g + b
            pltpu.make_async_copy(hk_s.at[src_v.at[ci]], buf, sem).wait()
            for g16 in range(CK // 16):
                sl = pl.ds(g16 * 16, 16)
                si = src_v[ci, sl]
                di = dst_v[ci, sl]
                e = plsc.load_gather(als_v, [si]) + \
                    plsc.load_gather(ald_v, [di])
                e = jnp.maximum(e, 0.2 * e)
                eid = (row0 + ci) * CK + g16 * 16 + lax.iota(jnp.int32, 16)
                p = jnp.where(eid < E, jnp.exp(e), 0.0)
                p_v[sl] = p

            def scale_body(j, c2):
                pe = p_v[j]
                for q in range(HC // 16):
                    qs = pl.ds(q * 16, 16)
                    buf[j, qs] = buf[j, qs] * pe
                return c2

            lax.fori_loop(0, CK, scale_body, 0)
            pltpu.sync_copy(p_v, denk_s.at[dst_v.at[ci]], add=True)
            pltpu.sync_copy(buf, outk_s.at[dst_v.at[ci]], add=True)

            @pl.when(ci + 2 < n_chunks)
            def _():
                pltpu.async_copy(hk_s.at[src_v.at[ci + 2]], buf, sem)
        return carry

    lax.fori_loop(0, n_chunks // 2, pair_body, 0)


def _zero_fill(zf_v, zd_v):
    def zf_body(i, c):
        zf_v[i // 4, pl.ds((i % 4) * 16, 16)] = jnp.zeros((16,), _f32)
        return c

    lax.fori_loop(0, 160 * 4, zf_body, 0)

    def zd_body(i, c):
        zd_v[pl.ds(i * 16, 16)] = jnp.zeros((16,), _f32)
        return c

    lax.fori_loop(0, BN // 16, zd_body, 0)


_SC_SCRATCH = [
    pltpu.VMEM((NP,), _f32),
    pltpu.VMEM((NP,), _f32),
    pltpu.VMEM((CK, HC), _f32),
    pltpu.VMEM((CK, HC), _f32),
    pltpu.VMEM((CK,), _f32),
    pltpu.VMEM((160, HC), _f32),
    pltpu.VMEM((BN,), _f32),
    pltpu.VMEM_SHARED((NP, HC), _f32),
    pltpu.VMEM_SHARED((NP, HC), _f32),
    pltpu.VMEM_SHARED((NP,), _f32),
    pltpu.SemaphoreType.DMA,
    pltpu.SemaphoreType.DMA,
]


def _sc1_body(hT, alsT, aldT, srcp, dstp, out_hbm, den_hbm,
              src_v, dst_v, als_v, ald_v, rows0_v, rows1_v, p_v, zf_v, zd_v,
              hk_s, outk_s, denk_s, sem0, sem1):
    c = lax.axis_index("c")
    s = lax.axis_index("s")
    node_off = s * BN
    row0 = s * CH1
    _zero_fill(zf_v, zd_v)
    pltpu.sync_copy(srcp.at[pl.ds(row0, CH1), :], src_v)
    pltpu.sync_copy(dstp.at[pl.ds(row0, CH1), :], dst_v)

    def head_body(kk, carry):
        k = c * 4 + kk
        pltpu.sync_copy(hT.at[k, pl.ds(node_off, BN), :],
                        hk_s.at[pl.ds(node_off, BN), :])
        for z in range(4):
            pltpu.sync_copy(zf_v, outk_s.at[pl.ds(node_off + z * 160, 160), :])
        pltpu.sync_copy(zd_v, denk_s.at[pl.ds(node_off, BN)])
        pltpu.sync_copy(alsT.at[k], als_v)
        pltpu.sync_copy(aldT.at[k], ald_v)
        plsc.subcore_barrier()
        _edge_phase(hk_s, outk_s, denk_s, src_v, dst_v, als_v, ald_v,
                    rows0_v, rows1_v, p_v, row0, CH1, sem0, sem1)
        plsc.subcore_barrier()
        pltpu.sync_copy(outk_s.at[pl.ds(node_off, BN), :],
                        out_hbm.at[k, pl.ds(node_off, BN), :])
        pltpu.sync_copy(denk_s.at[pl.ds(node_off, BN)],
                        den_hbm.at[k, pl.ds(node_off, BN)])
        plsc.subcore_barrier()
        return carry

    lax.fori_loop(0, 4, head_body, 0)


@jax.jit
def _sc1(hT, alsT, aldT, srcp, dstp):
    mesh = plsc.VectorSubcoreMesh(core_axis_name="c", subcore_axis_name="s")
    return pl.kernel(
        _sc1_body,
        out_type=[
            jax.ShapeDtypeStruct((H1, NP, HC), _f32),
            jax.ShapeDtypeStruct((H1, NP), _f32),
        ],
        mesh=mesh,
        scratch_types=[
            pltpu.VMEM((CH1, CK), jnp.int32),
            pltpu.VMEM((CH1, CK), jnp.int32),
        ] + _SC_SCRATCH,
    )(hT, alsT, aldT, srcp, dstp)


def _sc2_body(h2p, als2, ald2, srcp, dstp, out_hbm, den_hbm,
              src_v, dst_v, als_v, ald_v, rows0_v, rows1_v, p_v, zf_v, zd_v,
              hk_s, outk_s, denk_s, sem0, sem1):
    c = lax.axis_index("c")
    s = lax.axis_index("s")
    node_off = s * BN
    w = c * 16 + s
    row0 = w * CH2
    _zero_fill(zf_v, zd_v)

    pltpu.sync_copy(h2p.at[pl.ds(node_off, BN), :],
                    hk_s.at[pl.ds(node_off, BN), :])
    for z in range(4):
        pltpu.sync_copy(zf_v, outk_s.at[pl.ds(node_off + z * 160, 160), :])
    pltpu.sync_copy(zd_v, denk_s.at[pl.ds(node_off, BN)])
    pltpu.sync_copy(als2.at[0], als_v)
    pltpu.sync_copy(ald2.at[0], ald_v)
    pltpu.sync_copy(srcp.at[pl.ds(row0, CH2), :], src_v)
    pltpu.sync_copy(dstp.at[pl.ds(row0, CH2), :], dst_v)
    plsc.subcore_barrier()
    _edge_phase(hk_s, outk_s, denk_s, src_v, dst_v, als_v, ald_v,
                rows0_v, rows1_v, p_v, row0, CH2, sem0, sem1)
    plsc.subcore_barrier()
    pltpu.sync_copy(outk_s.at[pl.ds(node_off, BN), :],
                    out_hbm.at[c, pl.ds(node_off, BN), :])
    pltpu.sync_copy(denk_s.at[pl.ds(node_off, BN)],
                    den_hbm.at[c, pl.ds(node_off, BN)])


@jax.jit
def _sc2(h2p, als2, ald2, srcp, dstp):
    mesh = plsc.VectorSubcoreMesh(core_axis_name="c", subcore_axis_name="s")
    return pl.kernel(
        _sc2_body,
        out_type=[
            jax.ShapeDtypeStruct((2, NP, HC), _f32),
            jax.ShapeDtypeStruct((2, NP), _f32),
        ],
        mesh=mesh,
        scratch_types=[
            pltpu.VMEM((CH2, CK), jnp.int32),
            pltpu.VMEM((CH2, CK), jnp.int32),
        ] + _SC_SCRATCH,
    )(h2p, als2, ald2, srcp, dstp)


def _tc2_body(o_ref, d_ref, b_ref, w_ref, as_ref, ad_ref,
              h_ref, als_ref, ald_ref):
    acc = jnp.zeros((BN, HC), _f32)
    for k in range(H1):
        t = o_ref[k] / (d_ref[k][:, None] + 1e-16) + b_ref[k][None, :]
        t = jnp.where(t > 0, t, jnp.exp(t) - 1.0)
        acc = acc + jnp.dot(t, w_ref[k], preferred_element_type=_f32)
    h_ref[...] = acc
    als_ref[0, :] = jnp.dot(acc, as_ref[0, :], preferred_element_type=_f32)
    ald_ref[0, :] = jnp.dot(acc, ad_ref[0, :], preferred_element_type=_f32)


@jax.jit
def _tc2(out1, den1, b1r, W2r, a_src2, a_dst2):
    return pl.pallas_call(
        _tc2_body,
        grid=(NB,),
        in_specs=[
            pl.BlockSpec((H1, BN, HC), lambda i: (0, i, 0)),
            pl.BlockSpec((H1, BN), lambda i: (0, i)),
            pl.BlockSpec((H1, HC), lambda i: (0, 0)),
            pl.BlockSpec((H1, HC, HC), lambda i: (0, 0, 0)),
            pl.BlockSpec((1, HC), lambda i: (0, 0)),
            pl.BlockSpec((1, HC), lambda i: (0, 0)),
        ],
        out_specs=[
            pl.BlockSpec((BN, HC), lambda i: (i, 0)),
            pl.BlockSpec((1, BN), lambda i: (0, i)),
            pl.BlockSpec((1, BN), lambda i: (0, i)),
        ],
        out_shape=[
            jax.ShapeDtypeStruct((NP, HC), _f32),
            jax.ShapeDtypeStruct((1, NP), _f32),
            jax.ShapeDtypeStruct((1, NP), _f32),
        ],
    )(out1, den1, b1r, W2r, a_src2, a_dst2)


def _tc3_body(o_ref, d_ref, b_ref, wl_ref, bl_ref, out_ref):
    o = o_ref[0] + o_ref[1]
    d = d_ref[0] + d_ref[1]
    h2 = o / (d[:, None] + 1e-16) + b_ref[0, :][None, :]
    h2 = jnp.where(h2 > 0, h2, jnp.exp(h2) - 1.0)
    lg = jnp.dot(h2, wl_ref[...], preferred_element_type=_f32)
    lg = lg + bl_ref[0, :][None, :]
    m = jnp.max(lg, axis=1, keepdims=True)
    ls = jnp.log(jnp.sum(jnp.exp(lg - m), axis=1, keepdims=True))
    out_ref[...] = lg - m - ls


@jax.jit
def _tc3(out2, den2, b2r, Wlp, blp):
    return pl.pallas_call(
        _tc3_body,
        grid=(NB,),
        in_specs=[
            pl.BlockSpec((2, BN, HC), lambda i: (0, i, 0)),
            pl.BlockSpec((2, BN), lambda i: (0, i)),
            pl.BlockSpec((1, HC), lambda i: (0, 0)),
            pl.BlockSpec((HC, 128), lambda i: (0, 0)),
            pl.BlockSpec((1, 128), lambda i: (0, 0)),
        ],
        out_specs=pl.BlockSpec((BN, 128), lambda i: (i, 0)),
        out_shape=jax.ShapeDtypeStruct((NP, 128), _f32),
    )(out2, den2, b2r, Wlp, blp)


def _pad_edges(v, ep):
    pad = ep - E
    fill = (jnp.arange(pad, dtype=jnp.int32) * 97) % N
    return jnp.concatenate([v, fill]).reshape(ep // CK, CK)


@jax.jit
def kernel(x, edge_index, W1, a_src1, a_dst1, b1, W2, a_src2, a_dst2, b2,
           Wl, bl):
    src = edge_index[0].astype(jnp.int32)
    dst = edge_index[1].astype(jnp.int32)
    src1 = _pad_edges(src, EP1)
    dst1 = _pad_edges(dst, EP1)
    src2 = _pad_edges(src, EP2)
    dst2 = _pad_edges(dst, EP2)

    xp = jnp.pad(x, ((0, NP - N), (0, 0)))
    hT, alsT, aldT = _tc1(xp, W1, a_src1, a_dst1)
    out1, den1 = _sc1(hT, alsT, aldT, src1, dst1)

    b1r = b1.reshape(H1, HC)
    W2r = W2.reshape(H1, HC, HC)
    h2p, als2, ald2 = _tc2(out1, den1, b1r, W2r, a_src2, a_dst2)
    out2, den2 = _sc2(h2p, als2, ald2, src2, dst2)

    b2r = b2.reshape(1, HC)
    Wlp = jnp.pad(Wl, ((0, 0), (0, 128 - NCLS)))
    blp = jnp.pad(bl, (0, 128 - NCLS), constant_values=-1e30).reshape(1, 128)
    logp = _tc3(out2, den2, b2r, Wlp, blp)
    return logp[:N, :NCLS]


# speedup vs baseline: 20.7964x; 20.7964x over previous
import jax
import jax.numpy as jnp
from jax import lax
from jax.experimental import pallas as pl
from jax.experimental.pallas import tpu as pltpu
from jax.experimental.pallas import tpu_sc as plsc

N = 10000
E = 320000
F_IN = 128
HC = 64
H1 = 8
NCLS = 40

NP = 10240
NB = 16
BN = NP // NB

CH1 = 160
CH2 = 80
CK = 128
EP = 16 * CH1 * CK

_f32 = jnp.float32


def _tc1_body(x_ref, w_ref, as_ref, ad_ref, h_ref, als_ref, ald_ref):
    hb = jnp.dot(x_ref[...], w_ref[0], preferred_element_type=_f32)
    h_ref[0] = hb
    als_ref[0, 0, :] = jnp.dot(hb, as_ref[0, 0, :], preferred_element_type=_f32)
    ald_ref[0, 0, :] = jnp.dot(hb, ad_ref[0, 0, :], preferred_element_type=_f32)


@jax.jit
def _tc1(xp, W1r, a_src1, a_dst1):
    return pl.pallas_call(
        _tc1_body,
        grid=(NB, H1),
        in_specs=[
            pl.BlockSpec((BN, F_IN), lambda i, k: (i, 0)),
            pl.BlockSpec((1, F_IN, HC), lambda i, k: (k, 0, 0)),
            pl.BlockSpec((1, 1, HC), lambda i, k: (k, 0, 0)),
            pl.BlockSpec((1, 1, HC), lambda i, k: (k, 0, 0)),
        ],
        out_specs=[
            pl.BlockSpec((1, BN, HC), lambda i, k: (k, i, 0)),
            pl.BlockSpec((1, 1, BN), lambda i, k: (k, 0, i)),
            pl.BlockSpec((1, 1, BN), lambda i, k: (k, 0, i)),
        ],
        out_shape=[
            jax.ShapeDtypeStruct((H1, NP, HC), _f32),
            jax.ShapeDtypeStruct((H1, 1, NP), _f32),
            jax.ShapeDtypeStruct((H1, 1, NP), _f32),
        ],
    )(xp, W1r, a_src1, a_dst1)


def _edge_phase(hflat, outk_s, denk_s, als_s, ald_s, src_v, dst_v,
                rows0_v, rows1_v, ihx_v, alsg_v, aldg_v, p_v, koff, row0,
                n_chunks, sem0, sem1):
    def _fetch(ci, slot, buf, sem):
        def i_body(g16, c):
            sl = pl.ds(g16 * 16, 16)
            ihx_v[slot, sl] = src_v[ci, sl] + koff
            return c

        lax.fori_loop(0, CK // 16, i_body, 0)
        pltpu.async_copy(hflat.at[ihx_v.at[slot]], buf, sem)

    _fetch(0, 0, rows0_v, sem0)
    _fetch(1, 1, rows1_v, sem1)

    def pair_body(g, carry):
        for b, buf, sem in ((0, rows0_v, sem0), (1, rows1_v, sem1)):
            ci = 2 * g + b
            pltpu.sync_copy(als_s.at[src_v.at[ci]], alsg_v)
            pltpu.sync_copy(ald_s.at[dst_v.at[ci]], aldg_v)
            pltpu.make_async_copy(hflat.at[ihx_v.at[b]], buf, sem).wait()

            def g_body(g16, c2):
                s0 = g16 * 16
                sl = pl.ds(s0, 16)
                e = alsg_v[sl] + aldg_v[sl]
                e = jnp.maximum(e, 0.2 * e)
                eid = (row0 + ci) * CK + s0 + lax.iota(jnp.int32, 16)
                p = jnp.where(eid < E, jnp.exp(e), 0.0)
                p_v[sl] = p
                for r in range(16):
                    pe = p[r]
                    for q in range(HC // 16):
                        qs = pl.ds(q * 16, 16)
                        buf[s0 + r, qs] = buf[s0 + r, qs] * pe
                return c2

            lax.fori_loop(0, CK // 16, g_body, 0)
            pltpu.sync_copy(p_v, denk_s.at[dst_v.at[ci]], add=True)
            pltpu.sync_copy(buf, outk_s.at[dst_v.at[ci]], add=True)

            @pl.when(ci + 2 < n_chunks)
            def _():
                _fetch(ci + 2, b, buf, sem)
        return carry

    lax.fori_loop(0, n_chunks // 2, pair_body, 0)


def _unpack_edges(epk, row0, nrows, src_v, dst_v):
    pltpu.sync_copy(epk.at[pl.ds(row0, nrows), :], src_v)

    def row_body(r, c):
        for g in range(CK // 16):
            sl = pl.ds(g * 16, 16)
            v = src_v[r, sl]
            dst_v[r, sl] = v & 16383
            src_v[r, sl] = lax.shift_right_logical(v, 14)
        return c

    lax.fori_loop(0, nrows, row_body, 0)


def _zero_fill(zf_v, zd_v):
    def zf_body(i, c):
        zf_v[i // 4, pl.ds((i % 4) * 16, 16)] = jnp.zeros((16,), _f32)
        return c

    lax.fori_loop(0, 160 * 4, zf_body, 0)

    def zd_body(i, c):
        zd_v[pl.ds(i * 16, 16)] = jnp.zeros((16,), _f32)
        return c

    lax.fori_loop(0, BN // 16, zd_body, 0)


_SC_SCRATCH = [
    pltpu.VMEM((CK, HC), _f32),
    pltpu.VMEM((CK, HC), _f32),
    pltpu.VMEM((2, CK), jnp.int32),
    pltpu.VMEM((CK,), _f32),
    pltpu.VMEM((CK,), _f32),
    pltpu.VMEM((CK,), _f32),
    pltpu.VMEM((160, HC), _f32),
    pltpu.VMEM((BN,), _f32),
    pltpu.VMEM_SHARED((NP, HC), _f32),
    pltpu.VMEM_SHARED((NP,), _f32),
    pltpu.VMEM_SHARED((NP,), _f32),
    pltpu.VMEM_SHARED((NP,), _f32),
    pltpu.SemaphoreType.DMA,
    pltpu.SemaphoreType.DMA,
]


def _sc1_body(hflat, alsT, aldT, epk, out_hbm, den_hbm,
              src_v, dst_v, rows0_v, rows1_v, ihx_v, alsg_v, aldg_v, p_v,
              zf_v, zd_v, outk_s, denk_s, als_s, ald_s, sem0, sem1):
    c = lax.axis_index("c")
    s = lax.axis_index("s")
    node_off = s * BN
    row0 = s * CH1
    nsl = pl.ds(node_off, BN)
    _zero_fill(zf_v, zd_v)
    _unpack_edges(epk, row0, CH1, src_v, dst_v)

    def head_body(kk, carry):
        k = c * 4 + kk
        for z in range(4):
            pltpu.sync_copy(zf_v, outk_s.at[pl.ds(node_off + z * 160, 160), :])
        pltpu.sync_copy(zd_v, denk_s.at[nsl])
        pltpu.sync_copy(alsT.at[k, 0, nsl], als_s.at[nsl])
        pltpu.sync_copy(aldT.at[k, 0, nsl], ald_s.at[nsl])
        plsc.subcore_barrier()
        _edge_phase(hflat, outk_s, denk_s, als_s, ald_s, src_v, dst_v,
                    rows0_v, rows1_v, ihx_v, alsg_v, aldg_v, p_v, k * NP,
                    row0, CH1, sem0, sem1)
        plsc.subcore_barrier()
        pltpu.sync_copy(outk_s.at[pl.ds(node_off, BN), :],
                        out_hbm.at[k, pl.ds(node_off, BN), :])
        pltpu.sync_copy(denk_s.at[pl.ds(node_off, BN)],
                        den_hbm.at[k, 0, pl.ds(node_off, BN)])
        plsc.subcore_barrier()
        return carry

    lax.fori_loop(0, 4, head_body, 0)


@jax.jit
def _sc1(hflat, alsT, aldT, epk):
    mesh = plsc.VectorSubcoreMesh(core_axis_name="c", subcore_axis_name="s")
    return pl.kernel(
        _sc1_body,
        out_type=[
            jax.ShapeDtypeStruct((H1, NP, HC), _f32),
            jax.ShapeDtypeStruct((H1, 1, NP), _f32),
        ],
        mesh=mesh,
        scratch_types=[
            pltpu.VMEM((CH1, CK), jnp.int32),
            pltpu.VMEM((CH1, CK), jnp.int32),
        ] + _SC_SCRATCH,
        compiler_params=pltpu.CompilerParams(use_tc_tiling_on_sc=False),
    )(hflat, alsT, aldT, epk)


def _sc2_body(h2p, als2, ald2, epk, out_hbm, den_hbm,
              src_v, dst_v, rows0_v, rows1_v, ihx_v, alsg_v, aldg_v, p_v,
              zf_v, zd_v, outk_s, denk_s, als_s, ald_s, sem0, sem1):
    c = lax.axis_index("c")
    s = lax.axis_index("s")
    node_off = s * BN
    w = c * 16 + s
    row0 = w * CH2
    nsl = pl.ds(node_off, BN)
    _zero_fill(zf_v, zd_v)

    for z in range(4):
        pltpu.sync_copy(zf_v, outk_s.at[pl.ds(node_off + z * 160, 160), :])
    pltpu.sync_copy(zd_v, denk_s.at[nsl])
    pltpu.sync_copy(als2.at[0, nsl], als_s.at[nsl])
    pltpu.sync_copy(ald2.at[0, nsl], ald_s.at[nsl])
    _unpack_edges(epk, row0, CH2, src_v, dst_v)
    plsc.subcore_barrier()
    _edge_phase(h2p, outk_s, denk_s, als_s, ald_s, src_v, dst_v,
                rows0_v, rows1_v, ihx_v, alsg_v, aldg_v, p_v, 0,
                row0, CH2, sem0, sem1)
    plsc.subcore_barrier()
    pltpu.sync_copy(outk_s.at[pl.ds(node_off, BN), :],
                    out_hbm.at[c, pl.ds(node_off, BN), :])
    pltpu.sync_copy(denk_s.at[pl.ds(node_off, BN)],
                    den_hbm.at[c, 0, pl.ds(node_off, BN)])


@jax.jit
def _sc2(h2p, als2, ald2, epk):
    mesh = plsc.VectorSubcoreMesh(core_axis_name="c", subcore_axis_name="s")
    return pl.kernel(
        _sc2_body,
        out_type=[
            jax.ShapeDtypeStruct((2, NP, HC), _f32),
            jax.ShapeDtypeStruct((2, 1, NP), _f32),
        ],
        mesh=mesh,
        scratch_types=[
            pltpu.VMEM((CH2, CK), jnp.int32),
            pltpu.VMEM((CH2, CK), jnp.int32),
        ] + _SC_SCRATCH,
        compiler_params=pltpu.CompilerParams(use_tc_tiling_on_sc=False),
    )(h2p, als2, ald2, epk)


def _tc2_body(o_ref, d_ref, b_ref, w_ref, as_ref, ad_ref,
              h_ref, als_ref, ald_ref):
    acc = jnp.zeros((BN, HC), _f32)
    for k in range(H1):
        t = o_ref[k] / (d_ref[k, 0][:, None] + 1e-16) + b_ref[k][None, :]
        t = jnp.where(t > 0, t, jnp.exp(t) - 1.0)
        acc = acc + jnp.dot(t, w_ref[k], preferred_element_type=_f32)
    h_ref[...] = acc
    als_ref[0, :] = jnp.dot(acc, as_ref[0, :], preferred_element_type=_f32)
    ald_ref[0, :] = jnp.dot(acc, ad_ref[0, :], preferred_element_type=_f32)


@jax.jit
def _tc2(out1, den1, b1r, W2r, a_src2, a_dst2):
    return pl.pallas_call(
        _tc2_body,
        grid=(NB,),
        in_specs=[
            pl.BlockSpec((H1, BN, HC), lambda i: (0, i, 0)),
            pl.BlockSpec((H1, 1, BN), lambda i: (0, 0, i)),
            pl.BlockSpec((H1, HC), lambda i: (0, 0)),
            pl.BlockSpec((H1, HC, HC), lambda i: (0, 0, 0)),
            pl.BlockSpec((1, HC), lambda i: (0, 0)),
            pl.BlockSpec((1, HC), lambda i: (0, 0)),
        ],
        out_specs=[
            pl.BlockSpec((BN, HC), lambda i: (i, 0)),
            pl.BlockSpec((1, BN), lambda i: (0, i)),
            pl.BlockSpec((1, BN), lambda i: (0, i)),
        ],
        out_shape=[
            jax.ShapeDtypeStruct((NP, HC), _f32),
            jax.ShapeDtypeStruct((1, NP), _f32),
            jax.ShapeDtypeStruct((1, NP), _f32),
        ],
    )(out1, den1, b1r, W2r, a_src2, a_dst2)


def _tc3_body(o_ref, d_ref, b_ref, wl_ref, bl_ref, out_ref):
    o = o_ref[0] + o_ref[1]
    d = d_ref[0, 0] + d_ref[1, 0]
    h2 = o / (d[:, None] + 1e-16) + b_ref[0, :][None, :]
    h2 = jnp.where(h2 > 0, h2, jnp.exp(h2) - 1.0)
    lg = jnp.dot(h2, wl_ref[...], preferred_element_type=_f32)
    lg = lg + bl_ref[0, :][None, :]
    m = jnp.max(lg, axis=1, keepdims=True)
    ls = jnp.log(jnp.sum(jnp.exp(lg - m), axis=1, keepdims=True))
    out_ref[...] = lg - m - ls


@jax.jit
def _tc3(out2, den2, b2r, Wlp, blp):
    return pl.pallas_call(
        _tc3_body,
        grid=(NB,),
        in_specs=[
            pl.BlockSpec((2, BN, HC), lambda i: (0, i, 0)),
            pl.BlockSpec((2, 1, BN), lambda i: (0, 0, i)),
            pl.BlockSpec((1, HC), lambda i: (0, 0)),
            pl.BlockSpec((HC, 128), lambda i: (0, 0)),
            pl.BlockSpec((1, 128), lambda i: (0, 0)),
        ],
        out_specs=pl.BlockSpec((BN, 128), lambda i: (i, 0)),
        out_shape=jax.ShapeDtypeStruct((NP, 128), _f32),
    )(out2, den2, b2r, Wlp, blp)


def _pad_edges(v, ep):
    pad = ep - E
    fill = (jnp.arange(pad, dtype=jnp.int32) * 97) % N
    return jnp.concatenate([v, fill]).reshape(ep // CK, CK)


@jax.jit
def kernel(x, edge_index, W1, a_src1, a_dst1, b1, W2, a_src2, a_dst2, b2,
           Wl, bl):
    src = edge_index[0].astype(jnp.int32)
    dst = edge_index[1].astype(jnp.int32)
    epk = _pad_edges(src, EP) * 16384 + _pad_edges(dst, EP)

    xp = jnp.pad(x, ((0, NP - N), (0, 0)))
    W1r = W1.reshape(F_IN, H1, HC).transpose(1, 0, 2)
    hT, alsT, aldT = _tc1(xp, W1r, a_src1.reshape(H1, 1, HC),
                          a_dst1.reshape(H1, 1, HC))
    out1, den1 = _sc1(hT.reshape(H1 * NP, HC), alsT, aldT, epk)

    b1r = b1.reshape(H1, HC)
    W2r = W2.reshape(H1, HC, HC)
    h2p, als2, ald2 = _tc2(out1, den1, b1r, W2r, a_src2, a_dst2)
    out2, den2 = _sc2(h2p, als2, ald2, epk)

    b2r = b2.reshape(1, HC)
    Wlp = jnp.pad(Wl, ((0, 0), (0, 128 - NCLS)))
    blp = jnp.pad(bl, (0, 128 - NCLS), constant_values=-1e30).reshape(1, 128)
    logp = _tc3(out2, den2, b2r, Wlp, blp)
    return logp[:N, :NCLS]


# speedup vs baseline: 29.2564x; 1.4068x over previous
import jax
import jax.numpy as jnp
from jax import lax
from jax.experimental import pallas as pl
from jax.experimental.pallas import tpu as pltpu
from jax.experimental.pallas import tpu_sc as plsc

N = 10000
E = 320000
F_IN = 128
HC = 64
H1 = 8
NCLS = 40

NP = 10240
NB = 16
BN = NP // NB

CKB = 256
EW1 = 20480
EW2 = 10240
EP = 16 * EW1

_f32 = jnp.float32


def _tc1_body(x_ref, w_ref, as_ref, ad_ref, h_ref, als_ref, ald_ref):
    hb = jnp.dot(x_ref[...], w_ref[0], preferred_element_type=_f32)
    h_ref[0] = hb
    als_ref[0, 0, :] = jnp.dot(hb, as_ref[0, 0, :], preferred_element_type=_f32)
    ald_ref[0, 0, :] = jnp.dot(hb, ad_ref[0, 0, :], preferred_element_type=_f32)


@jax.jit
def _tc1(xp, W1r, a_src1, a_dst1):
    return pl.pallas_call(
        _tc1_body,
        grid=(NB, H1),
        in_specs=[
            pl.BlockSpec((BN, F_IN), lambda i, k: (i, 0)),
            pl.BlockSpec((1, F_IN, HC), lambda i, k: (k, 0, 0)),
            pl.BlockSpec((1, 1, HC), lambda i, k: (k, 0, 0)),
            pl.BlockSpec((1, 1, HC), lambda i, k: (k, 0, 0)),
        ],
        out_specs=[
            pl.BlockSpec((1, BN, HC), lambda i, k: (k, i, 0)),
            pl.BlockSpec((1, 1, BN), lambda i, k: (k, 0, i)),
            pl.BlockSpec((1, 1, BN), lambda i, k: (k, 0, i)),
        ],
        out_shape=[
            jax.ShapeDtypeStruct((H1, NP, HC), _f32),
            jax.ShapeDtypeStruct((H1, 1, NP), _f32),
            jax.ShapeDtypeStruct((H1, 1, NP), _f32),
        ],
    )(xp, W1r, a_src1, a_dst1)


def _edge_phase(hflat, outk_s, denk_s, als_s, ald_s, src_v, dst_v,
                rows_v, ihx_v, alsg_v, aldg_v, p_v, koff, e0,
                n_chunks, semg, sems):
    def _fetch(ci, slot, buf, sem):
        def i_body(g16, c):
            sl = pl.ds(g16 * 16, 16)
            ihx_v[slot, sl] = src_v[pl.ds(ci * CKB + g16 * 16, 16)] + koff
            return c

        lax.fori_loop(0, CKB // 16, i_body, 0)
        pltpu.async_copy(hflat.at[ihx_v.at[slot]], buf, sem)

    rows0_v = rows_v.at[0]
    rows1_v = rows_v.at[1]
    sem0 = semg[0]
    sem1 = semg[1]
    _fetch(0, 0, rows0_v, sem0)
    _fetch(1, 1, rows1_v, sem1)

    def pair_body(g, carry):
        for b, buf, sem in ((0, rows0_v, sem0), (1, rows1_v, sem1)):
            ci = 2 * g + b
            esl = pl.ds(ci * CKB, CKB)
            pltpu.sync_copy(als_s.at[src_v.at[esl]], alsg_v.at[b])
            pltpu.sync_copy(ald_s.at[dst_v.at[esl]], aldg_v.at[b])
            pltpu.make_async_copy(hflat.at[ihx_v.at[b]], buf, sem).wait()

            def g_body(g16, c2):
                s0 = g16 * 16
                sl = pl.ds(s0, 16)
                e = alsg_v[b, sl] + aldg_v[b, sl]
                e = jnp.maximum(e, 0.2 * e)
                eid = e0 + ci * CKB + s0 + lax.iota(jnp.int32, 16)
                p = jnp.where(eid < E, jnp.exp(e), 0.0)
                p_v[b, sl] = p
                for r in range(16):
                    pe = p[r]
                    for q in range(HC // 16):
                        qs = pl.ds(q * 16, 16)
                        buf[s0 + r, qs] = buf[s0 + r, qs] * pe
                return c2

            lax.fori_loop(0, CKB // 16, g_body, 0)
            pltpu.sync_copy(p_v.at[b], denk_s.at[dst_v.at[esl]], add=True)
            pltpu.sync_copy(buf, outk_s.at[dst_v.at[esl]], add=True)

            @pl.when(ci + 2 < n_chunks)
            def _():
                _fetch(ci + 2, b, buf, sem)
        return carry

    lax.fori_loop(0, n_chunks // 2, pair_body, 0)


def _unpack_edges(epk, e0, ne, src_v, dst_v):
    pltpu.sync_copy(epk.at[pl.ds(e0, ne)], src_v)

    def u_body(i, c):
        sl = pl.ds(i * 16, 16)
        v = src_v[sl]
        dst_v[sl] = v & 16383
        src_v[sl] = lax.shift_right_logical(v, 14)
        return c

    lax.fori_loop(0, ne // 16, u_body, 0)


def _zero_fill(zf_v, zd_v):
    def zf_body(i, c):
        zf_v[i // 4, pl.ds((i % 4) * 16, 16)] = jnp.zeros((16,), _f32)
        return c

    lax.fori_loop(0, 160 * 4, zf_body, 0)

    def zd_body(i, c):
        zd_v[pl.ds(i * 16, 16)] = jnp.zeros((16,), _f32)
        return c

    lax.fori_loop(0, BN // 16, zd_body, 0)


_SC_SCRATCH = [
    pltpu.VMEM((2, CKB, HC), _f32),
    pltpu.VMEM((2, CKB), jnp.int32),
    pltpu.VMEM((2, CKB), _f32),
    pltpu.VMEM((2, CKB), _f32),
    pltpu.VMEM((2, CKB), _f32),
    pltpu.VMEM((160, HC), _f32),
    pltpu.VMEM((BN,), _f32),
    pltpu.VMEM_SHARED((NP, HC), _f32),
    pltpu.VMEM_SHARED((NP,), _f32),
    pltpu.VMEM_SHARED((NP,), _f32),
    pltpu.VMEM_SHARED((NP,), _f32),
] + [pltpu.SemaphoreType.DMA] * 2


def _sc1_body(hflat, alsT, aldT, epk, out_hbm, den_hbm,
              src_v, dst_v, rows_v, ihx_v, alsg_v, aldg_v, p_v,
              zf_v, zd_v, outk_s, denk_s, als_s, ald_s, *allsem):
    c = lax.axis_index("c")
    s = lax.axis_index("s")
    node_off = s * BN
    e0 = s * EW1
    nsl = pl.ds(node_off, BN)
    _zero_fill(zf_v, zd_v)
    _unpack_edges(epk, e0, EW1, src_v, dst_v)

    def head_body(kk, carry):
        k = c * 4 + kk
        for z in range(4):
            pltpu.sync_copy(zf_v, outk_s.at[pl.ds(node_off + z * 160, 160), :])
        pltpu.sync_copy(zd_v, denk_s.at[nsl])
        pltpu.sync_copy(alsT.at[k, 0, nsl], als_s.at[nsl])
        pltpu.sync_copy(aldT.at[k, 0, nsl], ald_s.at[nsl])
        plsc.subcore_barrier()
        _edge_phase(hflat, outk_s, denk_s, als_s, ald_s, src_v, dst_v,
                    rows_v, ihx_v, alsg_v, aldg_v, p_v, k * NP,
                    e0, EW1 // CKB, allsem, allsem)
        plsc.subcore_barrier()
        pltpu.sync_copy(outk_s.at[pl.ds(node_off, BN), :],
                        out_hbm.at[k, pl.ds(node_off, BN), :])
        pltpu.sync_copy(denk_s.at[pl.ds(node_off, BN)],
                        den_hbm.at[k, 0, pl.ds(node_off, BN)])
        plsc.subcore_barrier()
        return carry

    lax.fori_loop(0, 4, head_body, 0)


@jax.jit
def _sc1(hflat, alsT, aldT, epk):
    mesh = plsc.VectorSubcoreMesh(core_axis_name="c", subcore_axis_name="s")
    return pl.kernel(
        _sc1_body,
        out_type=[
            jax.ShapeDtypeStruct((H1, NP, HC), _f32),
            jax.ShapeDtypeStruct((H1, 1, NP), _f32),
        ],
        mesh=mesh,
        scratch_types=[
            pltpu.VMEM((EW1,), jnp.int32),
            pltpu.VMEM((EW1,), jnp.int32),
        ] + _SC_SCRATCH,
        compiler_params=pltpu.CompilerParams(use_tc_tiling_on_sc=False),
    )(hflat, alsT, aldT, epk)


def _sc2_body(h2p, als2, ald2, epk, out_hbm, den_hbm,
              src_v, dst_v, rows_v, ihx_v, alsg_v, aldg_v, p_v,
              zf_v, zd_v, outk_s, denk_s, als_s, ald_s, *allsem):
    c = lax.axis_index("c")
    s = lax.axis_index("s")
    node_off = s * BN
    w = c * 16 + s
    e0 = w * EW2
    nsl = pl.ds(node_off, BN)
    _zero_fill(zf_v, zd_v)

    for z in range(4):
        pltpu.sync_copy(zf_v, outk_s.at[pl.ds(node_off + z * 160, 160), :])
    pltpu.sync_copy(zd_v, denk_s.at[nsl])
    pltpu.sync_copy(als2.at[0, nsl], als_s.at[nsl])
    pltpu.sync_copy(ald2.at[0, nsl], ald_s.at[nsl])
    _unpack_edges(epk, e0, EW2, src_v, dst_v)
    plsc.subcore_barrier()
    _edge_phase(h2p, outk_s, denk_s, als_s, ald_s, src_v, dst_v,
                rows_v, ihx_v, alsg_v, aldg_v, p_v, 0,
                e0, EW2 // CKB, allsem, allsem)
    plsc.subcore_barrier()
    pltpu.sync_copy(outk_s.at[pl.ds(node_off, BN), :],
                    out_hbm.at[c, pl.ds(node_off, BN), :])
    pltpu.sync_copy(denk_s.at[pl.ds(node_off, BN)],
                    den_hbm.at[c, 0, pl.ds(node_off, BN)])


@jax.jit
def _sc2(h2p, als2, ald2, epk):
    mesh = plsc.VectorSubcoreMesh(core_axis_name="c", subcore_axis_name="s")
    return pl.kernel(
        _sc2_body,
        out_type=[
            jax.ShapeDtypeStruct((2, NP, HC), _f32),
            jax.ShapeDtypeStruct((2, 1, NP), _f32),
        ],
        mesh=mesh,
        scratch_types=[
            pltpu.VMEM((EW2,), jnp.int32),
            pltpu.VMEM((EW2,), jnp.int32),
        ] + _SC_SCRATCH,
        compiler_params=pltpu.CompilerParams(use_tc_tiling_on_sc=False),
    )(h2p, als2, ald2, epk)


def _tc2_body(o_ref, d_ref, b_ref, w_ref, as_ref, ad_ref,
              h_ref, als_ref, ald_ref):
    acc = jnp.zeros((BN, HC), _f32)
    for k in range(H1):
        t = o_ref[k] / (d_ref[k, 0][:, None] + 1e-16) + b_ref[k][None, :]
        t = jnp.where(t > 0, t, jnp.exp(t) - 1.0)
        acc = acc + jnp.dot(t, w_ref[k], preferred_element_type=_f32)
    h_ref[...] = acc
    als_ref[0, :] = jnp.dot(acc, as_ref[0, :], preferred_element_type=_f32)
    ald_ref[0, :] = jnp.dot(acc, ad_ref[0, :], preferred_element_type=_f32)


@jax.jit
def _tc2(out1, den1, b1r, W2r, a_src2, a_dst2):
    return pl.pallas_call(
        _tc2_body,
        grid=(NB,),
        in_specs=[
            pl.BlockSpec((H1, BN, HC), lambda i: (0, i, 0)),
            pl.BlockSpec((H1, 1, BN), lambda i: (0, 0, i)),
            pl.BlockSpec((H1, HC), lambda i: (0, 0)),
            pl.BlockSpec((H1, HC, HC), lambda i: (0, 0, 0)),
            pl.BlockSpec((1, HC), lambda i: (0, 0)),
            pl.BlockSpec((1, HC), lambda i: (0, 0)),
        ],
        out_specs=[
            pl.BlockSpec((BN, HC), lambda i: (i, 0)),
            pl.BlockSpec((1, BN), lambda i: (0, i)),
            pl.BlockSpec((1, BN), lambda i: (0, i)),
        ],
        out_shape=[
            jax.ShapeDtypeStruct((NP, HC), _f32),
            jax.ShapeDtypeStruct((1, NP), _f32),
            jax.ShapeDtypeStruct((1, NP), _f32),
        ],
    )(out1, den1, b1r, W2r, a_src2, a_dst2)


def _tc3_body(o_ref, d_ref, b_ref, wl_ref, bl_ref, out_ref):
    o = o_ref[0] + o_ref[1]
    d = d_ref[0, 0] + d_ref[1, 0]
    h2 = o / (d[:, None] + 1e-16) + b_ref[0, :][None, :]
    h2 = jnp.where(h2 > 0, h2, jnp.exp(h2) - 1.0)
    lg = jnp.dot(h2, wl_ref[...], preferred_element_type=_f32)
    lg = lg + bl_ref[0, :][None, :]
    m = jnp.max(lg, axis=1, keepdims=True)
    ls = jnp.log(jnp.sum(jnp.exp(lg - m), axis=1, keepdims=True))
    out_ref[...] = lg - m - ls


@jax.jit
def _tc3(out2, den2, b2r, Wlp, blp):
    return pl.pallas_call(
        _tc3_body,
        grid=(NB,),
        in_specs=[
            pl.BlockSpec((2, BN, HC), lambda i: (0, i, 0)),
            pl.BlockSpec((2, 1, BN), lambda i: (0, 0, i)),
            pl.BlockSpec((1, HC), lambda i: (0, 0)),
            pl.BlockSpec((HC, 128), lambda i: (0, 0)),
            pl.BlockSpec((1, 128), lambda i: (0, 0)),
        ],
        out_specs=pl.BlockSpec((BN, 128), lambda i: (i, 0)),
        out_shape=jax.ShapeDtypeStruct((NP, 128), _f32),
    )(out2, den2, b2r, Wlp, blp)


def _pad_edges(v, ep):
    pad = ep - E
    fill = (jnp.arange(pad, dtype=jnp.int32) * 97) % N
    return jnp.concatenate([v, fill])


@jax.jit
def kernel(x, edge_index, W1, a_src1, a_dst1, b1, W2, a_src2, a_dst2, b2,
           Wl, bl):
    src = edge_index[0].astype(jnp.int32)
    dst = edge_index[1].astype(jnp.int32)
    epk = _pad_edges(src, EP) * 16384 + _pad_edges(dst, EP)

    xp = jnp.pad(x, ((0, NP - N), (0, 0)))
    W1r = W1.reshape(F_IN, H1, HC).transpose(1, 0, 2)
    hT, alsT, aldT = _tc1(xp, W1r, a_src1.reshape(H1, 1, HC),
                          a_dst1.reshape(H1, 1, HC))
    out1, den1 = _sc1(hT.reshape(H1 * NP, HC), alsT, aldT, epk)

    b1r = b1.reshape(H1, HC)
    W2r = W2.reshape(H1, HC, HC)
    h2p, als2, ald2 = _tc2(out1, den1, b1r, W2r, a_src2, a_dst2)
    out2, den2 = _sc2(h2p, als2, ald2, epk)

    b2r = b2.reshape(1, HC)
    Wlp = jnp.pad(Wl, ((0, 0), (0, 128 - NCLS)))
    blp = jnp.pad(bl, (0, 128 - NCLS), constant_values=-1e30).reshape(1, 128)
    logp = _tc3(out2, den2, b2r, Wlp, blp)
    return logp[:N, :NCLS]
